# baseline (speedup 1.0000x reference)
import jax
import jax.numpy as jnp
from jax.experimental import pallas as pl

_RADIUS2 = 1.0
_CC1 = 256
_CAP = 256
_FBC = 512


def _prep_queries(p_ref, bi):
    xyz = p_ref[bi][:, 0:3]
    qt = jnp.transpose(xyz, (1, 0))
    x0 = qt[0, :][None, :]
    x1 = qt[1, :][None, :]
    x2 = qt[2, :][None, :]
    sq = x0 * x0 + x1 * x1 + x2 * x2
    return sq, qt.astype(jnp.bfloat16)


def _chunk_scan(p_ref, bi, sq_q, qt16, c, cc, n):
    xc = p_ref[bi, pl.ds(c, cc), 0:3]
    x0c = xc[:, 0][:, None]
    x1c = xc[:, 1][:, None]
    x2c = xc[:, 2][:, None]
    sq_c = x0c * x0c + x1c * x1c + x2c * x2c
    dot = jax.lax.dot_general(
        xc.astype(jnp.bfloat16), qt16,
        (((1,), (0,)), ((), ())),
        preferred_element_type=jnp.float32)
    d2 = (sq_q + sq_c) - 2.0 * dot
    col = jax.lax.broadcasted_iota(jnp.int32, (cc, 1), 0) + c
    enc = jnp.where(d2 < _RADIUS2, col, n)
    return jnp.min(enc, axis=0, keepdims=True)


def _phase1_kernel(p_ref, out_ref):
    nb, n, _ = p_ref.shape
    for bi in range(nb):
        sq_q, qt16 = _prep_queries(p_ref, bi)
        out_ref[bi] = _chunk_scan(p_ref, bi, sq_q, qt16, 0, _CC1, n)


def _phase2_kernel(p_ref, xg_ref, idx_ref, best1_ref, out_ref):
    nb, n, _ = p_ref.shape
    lane_col = jax.lax.broadcasted_iota(jnp.int32, (1, n), 1)
    merged = []
    for bi in range(nb):
        gt = jnp.transpose(xg_ref[bi], (1, 0))
        g0 = gt[0, :][None, :]
        g1 = gt[1, :][None, :]
        g2 = gt[2, :][None, :]
        sq_g = g0 * g0 + g1 * g1 + g2 * g2
        gt16 = gt.astype(jnp.bfloat16)
        xcand = p_ref[bi][:, 0:3]
        c0 = xcand[:, 0][:, None]
        c1 = xcand[:, 1][:, None]
        c2 = xcand[:, 2][:, None]
        sq_c = c0 * c0 + c1 * c1 + c2 * c2
        dot = jax.lax.dot_general(
            xcand.astype(jnp.bfloat16), gt16,
            (((1,), (0,)), ((), ())),
            preferred_element_type=jnp.float32)
        d2 = (sq_g + sq_c) - 2.0 * dot
        row = jax.lax.broadcasted_iota(jnp.int32, (n, 1), 0)
        enc = jnp.where(d2 < _RADIUS2, row, n)
        first_g = jnp.min(enc, axis=0, keepdims=True)
        val = jnp.where(first_g == n, 0, first_g)
        val_s = jnp.transpose(val, (1, 0))
        oh = idx_ref[bi] == lane_col
        writes = jnp.where(oh, val_s, n)
        scat = jnp.min(writes, axis=0, keepdims=True)
        b1 = best1_ref[bi]
        merged.append(jnp.where(scat != n, scat, b1))

    unfound = merged[0] == n
    for bi in range(1, nb):
        unfound = jnp.logical_or(unfound, merged[bi] == n)

    def fallback(_):
        outs = []
        for bi in range(nb):
            sq_q, qt16 = _prep_queries(p_ref, bi)
            best = jnp.full((1, n), n, jnp.int32)
            for k in range(n // _FBC):
                best = jnp.minimum(
                    best,
                    _chunk_scan(p_ref, bi, sq_q, qt16, k * _FBC, _FBC, n))
            outs.append(best)
        return tuple(outs)

    merged = jax.lax.cond(jnp.any(unfound), fallback,
                          lambda _: tuple(merged), None)
    for bi in range(nb):
        out_ref[bi] = jnp.where(merged[bi] == n, 0, merged[bi])


def kernel(p):
    b, n, ch = p.shape
    best1 = pl.pallas_call(
        _phase1_kernel,
        in_specs=[pl.BlockSpec((b, n, ch), lambda: (0, 0, 0))],
        out_specs=pl.BlockSpec((b, 1, n), lambda: (0, 0, 0)),
        out_shape=jax.ShapeDtypeStruct((b, 1, n), jnp.int32),
    )(p)

    flags = best1[:, 0, :] == n
    dst = jnp.where(flags, jnp.cumsum(flags.astype(jnp.int32), axis=1) - 1,
                    _CAP)
    col = jnp.broadcast_to(
        jnp.arange(n, dtype=jnp.int32)[None, :], (b, n))
    idx = jnp.zeros((b, _CAP), jnp.int32).at[
        jnp.arange(b)[:, None], dst].set(col, mode='drop')
    xg = jnp.take_along_axis(p[:, :, 0:3], idx[:, :, None], axis=1)

    out = pl.pallas_call(
        _phase2_kernel,
        in_specs=[
            pl.BlockSpec((b, n, ch), lambda: (0, 0, 0)),
            pl.BlockSpec((b, _CAP, 3), lambda: (0, 0, 0)),
            pl.BlockSpec((b, _CAP, 1), lambda: (0, 0, 0)),
            pl.BlockSpec((b, 1, n), lambda: (0, 0, 0)),
        ],
        out_specs=pl.BlockSpec((b, 1, n), lambda: (0, 0, 0)),
        out_shape=jax.ShapeDtypeStruct((b, 1, n), jnp.int32),
    )(p, xg, idx.reshape(b, _CAP, 1), best1)
    return out.reshape(b, n, 1)


# speedup vs baseline: 1.0020x; 1.0020x over previous
import jax
import jax.numpy as jnp
from jax.experimental import pallas as pl

_RADIUS2 = 1.0
_CC1 = 256
_CAP = 256
_FBC = 512


def _prep_queries(p_ref, bi):
    xyz = p_ref[bi][:, 0:3]
    qt = jnp.transpose(xyz, (1, 0))
    x0 = qt[0, :][None, :]
    x1 = qt[1, :][None, :]
    x2 = qt[2, :][None, :]
    sq = x0 * x0 + x1 * x1 + x2 * x2
    return sq, qt.astype(jnp.bfloat16)


def _chunk_scan(p_ref, bi, sq_q, qt16, c, cc, n):
    xc = p_ref[bi, pl.ds(c, cc), 0:3]
    x0c = xc[:, 0][:, None]
    x1c = xc[:, 1][:, None]
    x2c = xc[:, 2][:, None]
    sq_c = x0c * x0c + x1c * x1c + x2c * x2c
    dot = jax.lax.dot_general(
        xc.astype(jnp.bfloat16), qt16,
        (((1,), (0,)), ((), ())),
        preferred_element_type=jnp.float32)
    d2 = (sq_q + sq_c) - 2.0 * dot
    col = jax.lax.broadcasted_iota(jnp.int32, (cc, 1), 0) + c
    enc = jnp.where(d2 < _RADIUS2, col, n)
    return jnp.min(enc, axis=0, keepdims=True)


def _phase1_kernel(p_ref, out_ref):
    nb, n, _ = p_ref.shape
    for bi in range(nb):
        sq_q, qt16 = _prep_queries(p_ref, bi)
        out_ref[bi] = _chunk_scan(p_ref, bi, sq_q, qt16, 0, _CC1, n)


def _phase2_kernel(p_ref, xg_ref, idx_ref, best1_ref, out_ref):
    nb, n, _ = p_ref.shape
    lane_col = jax.lax.broadcasted_iota(jnp.int32, (1, n), 1)
    merged = []
    for bi in range(nb):
        gt = jnp.transpose(xg_ref[bi], (1, 0))
        g0 = gt[0, :][None, :]
        g1 = gt[1, :][None, :]
        g2 = gt[2, :][None, :]
        sq_g = g0 * g0 + g1 * g1 + g2 * g2
        gt16 = gt.astype(jnp.bfloat16)
        xcand = p_ref[bi][:, 0:3]
        c0 = xcand[:, 0][:, None]
        c1 = xcand[:, 1][:, None]
        c2 = xcand[:, 2][:, None]
        sq_c = c0 * c0 + c1 * c1 + c2 * c2
        dot = jax.lax.dot_general(
            xcand.astype(jnp.bfloat16), gt16,
            (((1,), (0,)), ((), ())),
            preferred_element_type=jnp.float32)
        d2 = (sq_g + sq_c) - 2.0 * dot
        row = jax.lax.broadcasted_iota(jnp.int32, (n, 1), 0)
        enc = jnp.where(d2 < _RADIUS2, row, n)
        first_g = jnp.min(enc, axis=0, keepdims=True)
        val = jnp.where(first_g == n, 0, first_g)
        val_s = jnp.transpose(val, (1, 0))
        oh = idx_ref[bi] == lane_col
        writes = jnp.where(oh, val_s, n)
        scat = jnp.min(writes, axis=0, keepdims=True)
        b1 = best1_ref[bi]
        merged.append(jnp.where(scat != n, scat, b1))

    unfound = merged[0] == n
    for bi in range(1, nb):
        unfound = jnp.logical_or(unfound, merged[bi] == n)
    for bi in range(nb):
        out_ref[bi] = jnp.where(merged[bi] == n, 0, merged[bi])

    @pl.when(jnp.any(unfound))
    def _fallback():
        for bi in range(nb):
            sq_q, qt16 = _prep_queries(p_ref, bi)
            best = jnp.full((1, n), n, jnp.int32)
            for k in range(n // _FBC):
                best = jnp.minimum(
                    best,
                    _chunk_scan(p_ref, bi, sq_q, qt16, k * _FBC, _FBC, n))
            out_ref[bi] = jnp.where(best == n, 0, best)


def kernel(p):
    b, n, ch = p.shape
    best1 = pl.pallas_call(
        _phase1_kernel,
        in_specs=[pl.BlockSpec((b, n, ch), lambda: (0, 0, 0))],
        out_specs=pl.BlockSpec((b, 1, n), lambda: (0, 0, 0)),
        out_shape=jax.ShapeDtypeStruct((b, 1, n), jnp.int32),
    )(p)

    flags = best1[:, 0, :] == n
    dst = jnp.where(flags, jnp.cumsum(flags.astype(jnp.int32), axis=1) - 1,
                    _CAP)
    col = jnp.broadcast_to(
        jnp.arange(n, dtype=jnp.int32)[None, :], (b, n))
    idx = jnp.zeros((b, _CAP), jnp.int32).at[
        jnp.arange(b)[:, None], dst].set(col, mode='drop')
    xg = jnp.take_along_axis(p[:, :, 0:3], idx[:, :, None], axis=1)

    out = pl.pallas_call(
        _phase2_kernel,
        in_specs=[
            pl.BlockSpec((b, n, ch), lambda: (0, 0, 0)),
            pl.BlockSpec((b, _CAP, 3), lambda: (0, 0, 0)),
            pl.BlockSpec((b, _CAP, 1), lambda: (0, 0, 0)),
            pl.BlockSpec((b, 1, n), lambda: (0, 0, 0)),
        ],
        out_specs=pl.BlockSpec((b, 1, n), lambda: (0, 0, 0)),
        out_shape=jax.ShapeDtypeStruct((b, 1, n), jnp.int32),
    )(p, xg, idx.reshape(b, _CAP, 1), best1)
    return out.reshape(b, n, 1)


# speedup vs baseline: 1.1612x; 1.1588x over previous
import jax
import jax.numpy as jnp
from jax.experimental import pallas as pl

_RADIUS2 = 1.0
_CC1 = 256
_CAP = 256
_FBC = 512


def _prep_queries(p_ref, bi):
    xyz = p_ref[bi][:, 0:3]
    qt = jnp.transpose(xyz, (1, 0))
    x0 = qt[0, :][None, :]
    x1 = qt[1, :][None, :]
    x2 = qt[2, :][None, :]
    sq = x0 * x0 + x1 * x1 + x2 * x2
    return sq, qt.astype(jnp.bfloat16)


def _chunk_scan(p_ref, bi, sq_q, qt16, c, cc, n):
    xc = p_ref[bi, pl.ds(c, cc), 0:3]
    x0c = xc[:, 0][:, None]
    x1c = xc[:, 1][:, None]
    x2c = xc[:, 2][:, None]
    sq_c = x0c * x0c + x1c * x1c + x2c * x2c
    dot = jax.lax.dot_general(
        xc.astype(jnp.bfloat16), qt16,
        (((1,), (0,)), ((), ())),
        preferred_element_type=jnp.float32)
    d2 = (sq_q + sq_c) - 2.0 * dot
    col = jax.lax.broadcasted_iota(jnp.int32, (cc, 1), 0) + c
    enc = jnp.where(d2 < _RADIUS2, col, n)
    return jnp.min(enc, axis=0, keepdims=True)


def _phase1_kernel(p_ref, out_ref):
    nb, n, _ = p_ref.shape
    for bi in range(nb):
        sq_q, qt16 = _prep_queries(p_ref, bi)
        out_ref[bi] = _chunk_scan(p_ref, bi, sq_q, qt16, 0, _CC1, n)


def _phase2_kernel(p_ref, xg_ref, idx_ref, best1_ref, out_ref):
    nb, n, _ = p_ref.shape
    lane_col = jax.lax.broadcasted_iota(jnp.int32, (1, n), 1)
    merged = []
    for bi in range(nb):
        sq_q, qt16 = _prep_queries(p_ref, bi)
        xg = xg_ref[bi]
        g0 = xg[:, 0][:, None]
        g1 = xg[:, 1][:, None]
        g2 = xg[:, 2][:, None]
        sq_g = g0 * g0 + g1 * g1 + g2 * g2
        dot = jax.lax.dot_general(
            xg.astype(jnp.bfloat16), qt16,
            (((1,), (0,)), ((), ())),
            preferred_element_type=jnp.float32)
        d2 = (sq_g + sq_q) - 2.0 * dot
        enc = jnp.where(d2 < _RADIUS2, lane_col, n)
        first_g = jnp.min(enc, axis=1, keepdims=True)
        val = jnp.where(first_g == n, 0, first_g)
        oh = idx_ref[bi] == lane_col
        writes = jnp.where(oh, val, n)
        scat = jnp.min(writes, axis=0, keepdims=True)
        b1 = best1_ref[bi]
        merged.append(jnp.where(scat != n, scat, b1))

    unfound = merged[0] == n
    for bi in range(1, nb):
        unfound = jnp.logical_or(unfound, merged[bi] == n)
    for bi in range(nb):
        out_ref[bi] = jnp.where(merged[bi] == n, 0, merged[bi])

    @pl.when(jnp.any(unfound))
    def _fallback():
        for bi in range(nb):
            sq_q, qt16 = _prep_queries(p_ref, bi)
            best = jnp.full((1, n), n, jnp.int32)
            for k in range(n // _FBC):
                best = jnp.minimum(
                    best,
                    _chunk_scan(p_ref, bi, sq_q, qt16, k * _FBC, _FBC, n))
            out_ref[bi] = jnp.where(best == n, 0, best)


def kernel(p):
    b, n, ch = p.shape
    best1 = pl.pallas_call(
        _phase1_kernel,
        in_specs=[pl.BlockSpec((b, n, ch), lambda: (0, 0, 0))],
        out_specs=pl.BlockSpec((b, 1, n), lambda: (0, 0, 0)),
        out_shape=jax.ShapeDtypeStruct((b, 1, n), jnp.int32),
    )(p)

    flags = best1[:, 0, :] == n
    dst = jnp.where(flags, jnp.cumsum(flags.astype(jnp.int32), axis=1) - 1,
                    _CAP)
    col = jnp.broadcast_to(
        jnp.arange(n, dtype=jnp.int32)[None, :], (b, n))
    idx = jnp.zeros((b, _CAP), jnp.int32).at[
        jnp.arange(b)[:, None], dst].set(col, mode='drop')
    xg = jnp.take_along_axis(p[:, :, 0:3], idx[:, :, None], axis=1)

    out = pl.pallas_call(
        _phase2_kernel,
        in_specs=[
            pl.BlockSpec((b, n, ch), lambda: (0, 0, 0)),
            pl.BlockSpec((b, _CAP, 3), lambda: (0, 0, 0)),
            pl.BlockSpec((b, _CAP, 1), lambda: (0, 0, 0)),
            pl.BlockSpec((b, 1, n), lambda: (0, 0, 0)),
        ],
        out_specs=pl.BlockSpec((b, 1, n), lambda: (0, 0, 0)),
        out_shape=jax.ShapeDtypeStruct((b, 1, n), jnp.int32),
    )(p, xg, idx.reshape(b, _CAP, 1), best1)
    return out.reshape(b, n, 1)


# speedup vs baseline: 3.1194x; 2.6864x over previous
import jax
import jax.numpy as jnp
from jax.experimental import pallas as pl

_RADIUS2 = 1.0
_CC1 = 256
_CAP = 256
_FBC = 512


def _prep_queries(p_ref, bi):
    xyz = p_ref[bi][:, 0:3]
    qt = jnp.transpose(xyz, (1, 0))
    x0 = qt[0, :][None, :]
    x1 = qt[1, :][None, :]
    x2 = qt[2, :][None, :]
    sq = x0 * x0 + x1 * x1 + x2 * x2
    return sq, qt.astype(jnp.bfloat16)


def _chunk_scan(p_ref, bi, sq_q, qt16, c, cc, n):
    xc = p_ref[bi, pl.ds(c, cc), 0:3]
    x0c = xc[:, 0][:, None]
    x1c = xc[:, 1][:, None]
    x2c = xc[:, 2][:, None]
    sq_c = x0c * x0c + x1c * x1c + x2c * x2c
    dot = jax.lax.dot_general(
        xc.astype(jnp.bfloat16), qt16,
        (((1,), (0,)), ((), ())),
        preferred_element_type=jnp.float32)
    d2 = (sq_q + sq_c) - 2.0 * dot
    col = jax.lax.broadcasted_iota(jnp.int32, (cc, 1), 0) + c
    enc = jnp.where(d2 < _RADIUS2, col, n)
    return jnp.min(enc, axis=0, keepdims=True)


def _phase1_kernel(p_ref, out_ref):
    nb, n, _ = p_ref.shape
    for bi in range(nb):
        sq_q, qt16 = _prep_queries(p_ref, bi)
        out_ref[bi] = _chunk_scan(p_ref, bi, sq_q, qt16, 0, _CC1, n)


def _phase2_kernel(p_ref, idx_ref, best1_ref, out_ref):
    nb, n, _ = p_ref.shape
    lane_col = jax.lax.broadcasted_iota(jnp.int32, (1, n), 1)
    merged = []
    for bi in range(nb):
        sq_q, qt16 = _prep_queries(p_ref, bi)
        oh = idx_ref[bi] == lane_col
        xg16 = jax.lax.dot_general(
            oh.astype(jnp.bfloat16),
            p_ref[bi][:, 0:3].astype(jnp.bfloat16),
            (((1,), (0,)), ((), ())),
            preferred_element_type=jnp.float32).astype(jnp.bfloat16)
        sq_g = jnp.min(jnp.where(oh, sq_q, jnp.inf),
                       axis=1, keepdims=True)
        dot = jax.lax.dot_general(
            xg16, qt16,
            (((1,), (0,)), ((), ())),
            preferred_element_type=jnp.float32)
        d2 = (sq_g + sq_q) - 2.0 * dot
        enc = jnp.where(d2 < _RADIUS2, lane_col, n)
        first_g = jnp.min(enc, axis=1, keepdims=True)
        val = jnp.where(first_g == n, 0, first_g)
        writes = jnp.where(oh, val, n)
        scat = jnp.min(writes, axis=0, keepdims=True)
        b1 = best1_ref[bi]
        merged.append(jnp.where(scat != n, scat, b1))

    unfound = merged[0] == n
    for bi in range(1, nb):
        unfound = jnp.logical_or(unfound, merged[bi] == n)
    for bi in range(nb):
        out_ref[bi] = jnp.where(merged[bi] == n, 0, merged[bi])

    @pl.when(jnp.any(unfound))
    def _fallback():
        for bi in range(nb):
            sq_q, qt16 = _prep_queries(p_ref, bi)
            best = jnp.full((1, n), n, jnp.int32)
            for k in range(n // _FBC):
                best = jnp.minimum(
                    best,
                    _chunk_scan(p_ref, bi, sq_q, qt16, k * _FBC, _FBC, n))
            out_ref[bi] = jnp.where(best == n, 0, best)


def kernel(p):
    b, n, ch = p.shape
    best1 = pl.pallas_call(
        _phase1_kernel,
        in_specs=[pl.BlockSpec((b, n, ch), lambda: (0, 0, 0))],
        out_specs=pl.BlockSpec((b, 1, n), lambda: (0, 0, 0)),
        out_shape=jax.ShapeDtypeStruct((b, 1, n), jnp.int32),
    )(p)

    col = jnp.broadcast_to(jnp.arange(n, dtype=jnp.int32)[None, :], (b, n))
    keys = jnp.where(best1[:, 0, :] == n, col, n)
    idx = jnp.minimum(jnp.sort(keys, axis=1)[:, :_CAP], n - 1)

    out = pl.pallas_call(
        _phase2_kernel,
        in_specs=[
            pl.BlockSpec((b, n, ch), lambda: (0, 0, 0)),
            pl.BlockSpec((b, _CAP, 1), lambda: (0, 0, 0)),
            pl.BlockSpec((b, 1, n), lambda: (0, 0, 0)),
        ],
        out_specs=pl.BlockSpec((b, 1, n), lambda: (0, 0, 0)),
        out_shape=jax.ShapeDtypeStruct((b, 1, n), jnp.int32),
    )(p, idx.reshape(b, _CAP, 1), best1)
    return out.reshape(b, n, 1)
